# Initial kernel scaffold; baseline (speedup 1.0000x reference)
import jax
import jax.numpy as jnp
from jax.experimental import pallas as pl


def kernel(z, indices, output_size):
    raise NotImplementedError("write your pallas kernel here")


# speedup vs baseline: 30.8973x; 30.8973x over previous
import jax
import jax.numpy as jnp
from jax import lax
from jax.experimental import pallas as pl
from jax.experimental.pallas import tpu as pltpu
from jax.experimental.pallas import tpu_sc as plsc

B, C, HP, WP = 4, 96, 112, 112
HO, WO = 224, 224
PLANES = B * C
IN_WORDS = HP * WP * 2
OUT_WORDS = HO * WO * 2
HALF_IN = IN_WORDS // 2
HALF_OUT = OUT_WORDS // 2
NUM_HALVES = PLANES * 2
NC, NS, L = 2, 16, 16
NW = NC * NS
HALVES_PER_W = NUM_HALVES // NW


def _unpool_body(z_hbm, idx_hbm, off_hbm, out_hbm, vals_v, inds_v, offv_v, obuf_v):
    wid = lax.axis_index("s") * NC + lax.axis_index("c")
    pltpu.sync_copy(off_hbm, offv_v)
    off = offv_v[...]
    lane = lax.iota(jnp.int32, L)
    comp = lane & 1
    zeros = jnp.zeros((L,), jnp.float32)

    def plane_body(k, carry):
        h = wid * HALVES_PER_W + k
        plane = h // 2
        half = h & 1
        base_word = half * HALF_OUT
        pltpu.sync_copy(z_hbm.at[plane, pl.ds(half * HALF_IN, HALF_IN)], vals_v)
        pltpu.sync_copy(idx_hbm.at[plane, pl.ds(half * HALF_IN, HALF_IN)], inds_v)

        def zero_body(t, c):
            obuf_v[pl.ds(t * L, L)] = zeros
            return c

        lax.fori_loop(0, HALF_OUT // L, zero_body, 0)

        def scat_body(t, c):
            v = vals_v[pl.ds(t * L, L)]
            ix = inds_v[pl.ds(t * L, L)]
            local = (ix + off) * 2 + comp - base_word
            plsc.store_scatter(obuf_v, [local], v)
            return c

        lax.fori_loop(0, HALF_IN // L, scat_body, 0)
        pltpu.sync_copy(obuf_v, out_hbm.at[plane, pl.ds(base_word, HALF_OUT)])
        return carry

    lax.fori_loop(0, HALVES_PER_W, plane_body, 0)


def kernel(z, indices, output_size):
    zf = z.reshape(PLANES, IN_WORDS)
    idxf = indices.reshape(PLANES, IN_WORDS)
    off = jnp.broadcast_to(jnp.asarray(output_size, jnp.int32) - HO, (L,))
    mesh = plsc.VectorSubcoreMesh(core_axis_name="c", subcore_axis_name="s")
    out = pl.kernel(
        _unpool_body,
        out_type=jax.ShapeDtypeStruct((PLANES, OUT_WORDS), jnp.float32),
        mesh=mesh,
        compiler_params=pltpu.CompilerParams(needs_layout_passes=False),
        scratch_types=[
            pltpu.VMEM((HALF_IN,), jnp.float32),
            pltpu.VMEM((HALF_IN,), jnp.int32),
            pltpu.VMEM((L,), jnp.int32),
            pltpu.VMEM((HALF_OUT,), jnp.float32),
        ],
    )(zf, idxf, off)
    return out.reshape(B, C, HO, WO, 2)
